# baseline (speedup 1.0000x reference)
import functools

import jax
import jax.numpy as jnp
from jax import lax
from jax.experimental import pallas as pl
from jax.experimental.pallas import tpu as pltpu
from jax.experimental.pallas import tpu_sc as plsc

_B, _N, _D = 16, 4096, 512
_L = 16
_NJ = _D // _L
_R = 64
_HALF = _N // 2
_NCH = _HALF // _R
_PW = _D + _L


def _sc_mean(x_hbm, out_hbm, buf0, buf1, flag_v, part_v, tmp_v, res_v, shared,
             sem0, sem1):
    c = lax.axis_index("c")
    s = lax.axis_index("s")
    b = c * 8 + s // 2
    h = s % 2
    row_base = h * _HALF

    def chunk_copy(ci, buf, sem):
        return pltpu.make_async_copy(
            x_hbm.at[b, pl.ds(row_base + ci * _R, _R)], buf, sem)

    iota = lax.iota(jnp.int32, _L)
    zero = jnp.zeros((_L,), jnp.float32)
    init = tuple(zero for _ in range(_NJ + 1))

    def process(buf, carry):
        def row_body(r, cr):
            accs = cr[:_NJ]
            new = []
            nz = None
            for j in range(_NJ):
                v = buf[r, pl.ds(j * _L, _L)]
                new.append(accs[j] + v)
                m = v != 0.0
                nz = m if nz is None else (nz | m)
            flag_v[pl.ds(r * _L, _L)] = jnp.where(nz, 1.0, 0.0)
            return tuple(new) + (cr[_NJ],)

        carry = lax.fori_loop(0, _R, row_body, carry)

        def grp_body(g, cnt):
            anyf = None
            base = g * (_L * _L) + iota * _L
            for col in range(_L):
                fv = plsc.load_gather(flag_v, [base + col])
                anyf = fv if anyf is None else jnp.maximum(anyf, fv)
            return cnt + jnp.where(anyf > 0.0, 1.0, 0.0)

        cnt = lax.fori_loop(0, _R // _L, grp_body, carry[_NJ])
        return carry[:_NJ] + (cnt,)

    def chunk_body(i, carry):
        pltpu.sync_copy(x_hbm.at[b, pl.ds(row_base + i * _R, _R)], buf0)
        return process(buf0, carry)

    carry = lax.fori_loop(0, _NCH, chunk_body, init)

    accs, cntv = carry[:_NJ], carry[_NJ]
    for j in range(_NJ):
        part_v[pl.ds(j * _L, _L)] = accs[j]

    flag_v[pl.ds(0, _L)] = cntv
    tot = None
    for j in range(_L):
        lane = plsc.load_gather(flag_v, [jnp.full((_L,), j, jnp.int32)])
        tot = lane if tot is None else tot + lane
    part_v[pl.ds(_D, _L)] = tot

    pltpu.sync_copy(part_v, shared.at[s])
    plsc.subcore_barrier()

    @pl.when(h == 0)
    def _combine():
        pltpu.sync_copy(shared.at[s + 1], tmp_v)
        cnt_tot = part_v[pl.ds(_D, _L)] + tmp_v[pl.ds(_D, _L)]
        inv = 1.0 / cnt_tot
        for j in range(_NJ):
            sl = pl.ds(j * _L, _L)
            res_v[sl] = (part_v[sl] + tmp_v[sl]) * inv
        pltpu.sync_copy(res_v, out_hbm.at[b])


@functools.cache
def _build():
    return pl.kernel(
        _sc_mean,
        mesh=plsc.VectorSubcoreMesh(core_axis_name="c", subcore_axis_name="s",
                                    num_cores=2, num_subcores=16),
        out_type=jax.ShapeDtypeStruct((_B, _D), jnp.float32),
        compiler_params=pltpu.CompilerParams(needs_layout_passes=False,
                                             use_tc_tiling_on_sc=False),
        scratch_types=[
            pltpu.VMEM((_R, _D), jnp.float32),
            pltpu.VMEM((_R, _D), jnp.float32),
            pltpu.VMEM((_R * _L,), jnp.float32),
            pltpu.VMEM((_PW,), jnp.float32),
            pltpu.VMEM((_PW,), jnp.float32),
            pltpu.VMEM((_D,), jnp.float32),
            pltpu.VMEM_SHARED((16, _PW), jnp.float32),
            pltpu.SemaphoreType.DMA,
            pltpu.SemaphoreType.DMA,
        ],
    )


@jax.jit
def kernel(x):
    return _build()(x)


# speedup vs baseline: 1.4307x; 1.4307x over previous
import functools

import jax
import jax.numpy as jnp
from jax import lax
from jax.experimental import pallas as pl
from jax.experimental.pallas import tpu as pltpu
from jax.experimental.pallas import tpu_sc as plsc

_B, _N, _D = 16, 4096, 512
_L = 16
_NJ = _D // _L
_R = 64
_HALF = _N // 2
_NCH = _HALF // _R
_PW = _D + _L


def _sc_mean(x_hbm, out_hbm, buf0, buf1, flag_v, part_v, tmp_v, res_v, shared,
             sem0, sem1):
    c = lax.axis_index("c")
    s = lax.axis_index("s")
    b = c * 8 + s // 2
    h = s % 2
    row_base = h * _HALF

    def chunk_copy(ci, buf, sem):
        return pltpu.make_async_copy(
            x_hbm.at[b, pl.ds(row_base + ci * _R, _R)], buf, sem)

    iota = lax.iota(jnp.int32, _L)
    zero = jnp.zeros((_L,), jnp.float32)
    init = tuple(zero for _ in range(_NJ + 1))

    def process(buf, carry):
        def row_body(r, cr):
            accs = cr[:_NJ]
            new = []
            nz = None
            for j in range(_NJ):
                v = buf[r, pl.ds(j * _L, _L)]
                new.append(accs[j] + v)
                w = plsc.bitcast(v, jnp.int32)
                nz = w if nz is None else (nz | w)
            flag_v[pl.ds(r * _L, _L)] = nz << 1
            return tuple(new) + (cr[_NJ],)

        carry = lax.fori_loop(0, _R, row_body, carry)

        def grp_body(g, cnt):
            anyf = None
            base = g * (_L * _L) + iota * _L
            for col in range(_L):
                fv = plsc.load_gather(flag_v, [base + col])
                anyf = fv if anyf is None else (anyf | fv)
            return cnt + jnp.where(anyf != 0, 1.0, 0.0)

        cnt = lax.fori_loop(0, _R // _L, grp_body, carry[_NJ])
        return carry[:_NJ] + (cnt,)

    chunk_copy(0, buf0, sem0).start()

    def pair_body(i, carry):
        chunk_copy(2 * i + 1, buf1, sem1).start()
        chunk_copy(2 * i, buf0, sem0).wait()
        carry = process(buf0, carry)
        chunk_copy(jnp.minimum(2 * i + 2, _NCH - 1), buf0, sem0).start()
        chunk_copy(2 * i + 1, buf1, sem1).wait()
        return process(buf1, carry)

    carry = lax.fori_loop(0, _NCH // 2, pair_body, init)
    chunk_copy(_NCH - 1, buf0, sem0).wait()

    accs, cntv = carry[:_NJ], carry[_NJ]
    for j in range(_NJ):
        part_v[pl.ds(j * _L, _L)] = accs[j]

    part_v[pl.ds(_D, _L)] = cntv
    tot = None
    for j in range(_L):
        lane = plsc.load_gather(part_v, [jnp.full((_L,), _D + j, jnp.int32)])
        tot = lane if tot is None else tot + lane
    part_v[pl.ds(_D, _L)] = tot

    pltpu.sync_copy(part_v, shared.at[s])
    plsc.subcore_barrier()

    @pl.when(h == 0)
    def _combine():
        pltpu.sync_copy(shared.at[s + 1], tmp_v)
        cnt_tot = part_v[pl.ds(_D, _L)] + tmp_v[pl.ds(_D, _L)]
        inv = 1.0 / cnt_tot
        for j in range(_NJ):
            sl = pl.ds(j * _L, _L)
            res_v[sl] = (part_v[sl] + tmp_v[sl]) * inv
        pltpu.sync_copy(res_v, out_hbm.at[b])


@functools.cache
def _build():
    return pl.kernel(
        _sc_mean,
        mesh=plsc.VectorSubcoreMesh(core_axis_name="c", subcore_axis_name="s",
                                    num_cores=2, num_subcores=16),
        out_type=jax.ShapeDtypeStruct((_B, _D), jnp.float32),
        compiler_params=pltpu.CompilerParams(needs_layout_passes=False,
                                             use_tc_tiling_on_sc=False),
        scratch_types=[
            pltpu.VMEM((_R, _D), jnp.float32),
            pltpu.VMEM((_R, _D), jnp.float32),
            pltpu.VMEM((_R * _L,), jnp.int32),
            pltpu.VMEM((_PW,), jnp.float32),
            pltpu.VMEM((_PW,), jnp.float32),
            pltpu.VMEM((_D,), jnp.float32),
            pltpu.VMEM_SHARED((16, _PW), jnp.float32),
            pltpu.SemaphoreType.DMA,
            pltpu.SemaphoreType.DMA,
        ],
    )


@jax.jit
def kernel(x):
    return _build()(x)


# speedup vs baseline: 1.4862x; 1.0388x over previous
import functools

import jax
import jax.numpy as jnp
from jax import lax
from jax.experimental import pallas as pl
from jax.experimental.pallas import tpu as pltpu
from jax.experimental.pallas import tpu_sc as plsc

_B, _N, _D = 16, 4096, 512
_L = 16
_NJ = _D // _L
_R = 64
_PW = _D + _L

_KS = 4
_KT = _B - _KS
_WPS = 32 // _KS
_RPW = _N // _WPS
_NCH = _RPW // _R

_CHUNK = 1024
_NC = _N // _CHUNK


def _sc_mean(x_hbm, out_hbm, buf0, buf1, flag_v, part_v, tmp_v, res_v, shared,
             sem0, sem1):
    c = lax.axis_index("c")
    s = lax.axis_index("s")
    g = c * 16 + s
    m = g // _WPS
    q = g % _WPS
    row_base = q * _RPW

    def chunk_copy(ci, buf, sem):
        return pltpu.make_async_copy(
            x_hbm.at[m, pl.ds(row_base + ci * _R, _R)], buf, sem)

    iota = lax.iota(jnp.int32, _L)
    zero = jnp.zeros((_L,), jnp.float32)
    init = tuple(zero for _ in range(_NJ + 1))

    def process(buf, carry):
        def row_body(r, cr):
            accs = cr[:_NJ]
            new = []
            nz = None
            for j in range(_NJ):
                v = buf[r, pl.ds(j * _L, _L)]
                new.append(accs[j] + v)
                w = plsc.bitcast(v, jnp.int32)
                nz = w if nz is None else (nz | w)
            flag_v[pl.ds(r * _L, _L)] = nz << 1
            return tuple(new) + (cr[_NJ],)

        carry = lax.fori_loop(0, _R, row_body, carry)

        def grp_body(gi, cnt):
            anyf = None
            base = gi * (_L * _L) + iota * _L
            for col in range(_L):
                fv = plsc.load_gather(flag_v, [base + col])
                anyf = fv if anyf is None else (anyf | fv)
            return cnt + jnp.where(anyf != 0, 1.0, 0.0)

        cnt = lax.fori_loop(0, _R // _L, grp_body, carry[_NJ])
        return carry[:_NJ] + (cnt,)

    chunk_copy(0, buf0, sem0).start()

    def pair_body(i, carry):
        chunk_copy(2 * i + 1, buf1, sem1).start()
        chunk_copy(2 * i, buf0, sem0).wait()
        carry = process(buf0, carry)
        chunk_copy(jnp.minimum(2 * i + 2, _NCH - 1), buf0, sem0).start()
        chunk_copy(2 * i + 1, buf1, sem1).wait()
        return process(buf1, carry)

    carry = lax.fori_loop(0, _NCH // 2, pair_body, init)
    chunk_copy(_NCH - 1, buf0, sem0).wait()

    accs, cntv = carry[:_NJ], carry[_NJ]
    for j in range(_NJ):
        part_v[pl.ds(j * _L, _L)] = accs[j]

    part_v[pl.ds(_D, _L)] = cntv
    tot = None
    for j in range(_L):
        lane = plsc.load_gather(part_v, [jnp.full((_L,), _D + j, jnp.int32)])
        tot = lane if tot is None else tot + lane
    part_v[pl.ds(_D, _L)] = tot

    pltpu.sync_copy(part_v, shared.at[s])
    plsc.subcore_barrier()

    @pl.when(q == 0)
    def _combine():
        acc = [part_v[pl.ds(j * _L, _L)] for j in range(_NJ)]
        cnt = part_v[pl.ds(_D, _L)]
        for p in range(1, _WPS):
            pltpu.sync_copy(shared.at[s + p], tmp_v)
            for j in range(_NJ):
                acc[j] = acc[j] + tmp_v[pl.ds(j * _L, _L)]
            cnt = cnt + tmp_v[pl.ds(_D, _L)]
        inv = 1.0 / cnt
        for j in range(_NJ):
            res_v[pl.ds(j * _L, _L)] = acc[j] * inv
        pltpu.sync_copy(res_v, out_hbm.at[m])


@functools.cache
def _build_sc():
    return pl.kernel(
        _sc_mean,
        mesh=plsc.VectorSubcoreMesh(core_axis_name="c", subcore_axis_name="s",
                                    num_cores=2, num_subcores=16),
        out_type=jax.ShapeDtypeStruct((_KS, _D), jnp.float32),
        compiler_params=pltpu.CompilerParams(needs_layout_passes=False,
                                             use_tc_tiling_on_sc=False),
        scratch_types=[
            pltpu.VMEM((_R, _D), jnp.float32),
            pltpu.VMEM((_R, _D), jnp.float32),
            pltpu.VMEM((_R * _L,), jnp.int32),
            pltpu.VMEM((_PW,), jnp.float32),
            pltpu.VMEM((_PW,), jnp.float32),
            pltpu.VMEM((_D,), jnp.float32),
            pltpu.VMEM_SHARED((16, _PW), jnp.float32),
            pltpu.SemaphoreType.DMA,
            pltpu.SemaphoreType.DMA,
        ],
    )


def _tc_mean(x_ref, out_ref, cnt_ref):
    ci = pl.program_id(1)

    @pl.when(ci == 0)
    def _init():
        out_ref[...] = jnp.zeros_like(out_ref)
        cnt_ref[0, 0] = 0.0

    blk = x_ref[0]
    out_ref[0] += jnp.sum(blk, axis=0, keepdims=True)
    valid = jnp.any(blk != 0, axis=-1)
    cnt_ref[0, 0] += jnp.sum(valid.astype(jnp.float32))

    @pl.when(ci == _NC - 1)
    def _finish():
        out_ref[...] = out_ref[...] / cnt_ref[0, 0]


@jax.jit
def kernel(x):
    y_sc = _build_sc()(x)
    y_tc = pl.pallas_call(
        _tc_mean,
        grid=(_KT, _NC),
        in_specs=[pl.BlockSpec((1, _CHUNK, _D), lambda b, c: (b + _KS, c, 0))],
        out_specs=pl.BlockSpec((1, 1, _D), lambda b, c: (b, 0, 0)),
        out_shape=jax.ShapeDtypeStruct((_KT, 1, _D), jnp.float32),
        scratch_shapes=[pltpu.SMEM((1, 1), jnp.float32)],
    )(x)
    return jnp.concatenate([y_sc, y_tc[:, 0, :]], axis=0)


# speedup vs baseline: 3.3191x; 2.2333x over previous
import functools

import jax
import jax.numpy as jnp
from jax import lax
from jax.experimental import pallas as pl
from jax.experimental.pallas import tpu as pltpu
from jax.experimental.pallas import tpu_sc as plsc

_B, _N, _D = 16, 4096, 512
_L = 16
_NJ = _D // _L
_R = 64
_TR = _R // 8

_KS = 4
_KT = _B - _KS
_WPS = 32 // _KS
_RPW = _N // _WPS
_NCH = _RPW // _R

_CHUNK = 1024
_NC = _N // _CHUNK


def _sc_mean(x_hbm, out_hbm, buf0, buf1, flag8, part_v, tmp_v, res_v, shared,
             sem0, sem1):
    c = lax.axis_index("c")
    s = lax.axis_index("s")
    g = c * 16 + s
    m = g // _WPS
    q = g % _WPS
    row_base = q * _RPW

    def chunk_copy(ci, buf, sem):
        return pltpu.make_async_copy(
            x_hbm.at[m, pl.ds(row_base + ci * _R, _R)], buf, sem)

    iota = lax.iota(jnp.int32, _L)
    idx8 = (iota % 8) * _L
    zero = jnp.zeros((_L,), jnp.float32)
    init = tuple(zero for _ in range(_NJ + 1))

    def process(buf, carry):
        def tr_body(tr, cr):
            accs = list(cr[:_NJ])
            for rr in range(8):
                nz = None
                for j in range(_NJ):
                    v = buf[tr * 8 + rr, pl.ds(j * _L, _L)]
                    accs[j] = accs[j] + v
                    w = plsc.bitcast(v, jnp.int32)
                    nz = w if nz is None else (nz | w)
                flag8[pl.ds(rr * _L, _L)] = nz << 1
            anyf = None
            for col in range(_L):
                fv = plsc.load_gather(flag8, [idx8 + col])
                anyf = fv if anyf is None else (anyf | fv)
            cnt = cr[_NJ] + jnp.where(anyf != 0, 1.0, 0.0)
            return tuple(accs) + (cnt,)

        return lax.fori_loop(0, _TR, tr_body, carry)

    chunk_copy(0, buf0, sem0).start()

    def pair_body(i, carry):
        chunk_copy(2 * i + 1, buf1, sem1).start()
        chunk_copy(2 * i, buf0, sem0).wait()
        carry = process(buf0, carry)
        chunk_copy(jnp.minimum(2 * i + 2, _NCH - 1), buf0, sem0).start()
        chunk_copy(2 * i + 1, buf1, sem1).wait()
        return process(buf1, carry)

    carry = lax.fori_loop(0, _NCH // 2, pair_body, init)
    chunk_copy(_NCH - 1, buf0, sem0).wait()

    accs, cntv = carry[:_NJ], carry[_NJ]
    for j in range(_NJ):
        part_v[j // 8, pl.ds((j % 8) * _L, _L)] = accs[j]

    flag8[pl.ds(0, _L)] = plsc.bitcast(cntv, jnp.int32)
    tot = None
    for j in range(_L):
        lane = plsc.load_gather(flag8, [jnp.full((_L,), j, jnp.int32)])
        lane = plsc.bitcast(lane, jnp.float32)
        tot = lane if tot is None else tot + lane
    part_v[4, pl.ds(0, _L)] = tot * 0.5

    pltpu.sync_copy(part_v, shared.at[pl.ds(s * 8, 8)])
    plsc.subcore_barrier()

    @pl.when(q == 0)
    def _combine():
        acc = [part_v[j // 8, pl.ds((j % 8) * _L, _L)] for j in range(_NJ)]
        cnt = part_v[4, pl.ds(0, _L)]
        for p in range(1, _WPS):
            pltpu.sync_copy(shared.at[pl.ds((s + p) * 8, 8)], tmp_v)
            for j in range(_NJ):
                acc[j] = acc[j] + tmp_v[j // 8, pl.ds((j % 8) * _L, _L)]
            cnt = cnt + tmp_v[4, pl.ds(0, _L)]
        inv = 1.0 / cnt
        for j in range(_NJ):
            res_v[j // 8, pl.ds((j % 8) * _L, _L)] = acc[j] * inv
        pltpu.sync_copy(res_v, out_hbm.at[m])


@functools.cache
def _build_sc():
    return pl.kernel(
        _sc_mean,
        mesh=plsc.VectorSubcoreMesh(core_axis_name="c", subcore_axis_name="s",
                                    num_cores=2, num_subcores=16),
        out_type=jax.ShapeDtypeStruct((_KS, 4, 128), jnp.float32),
        compiler_params=pltpu.CompilerParams(needs_layout_passes=False),
        scratch_types=[
            pltpu.VMEM((_R, _D), jnp.float32),
            pltpu.VMEM((_R, _D), jnp.float32),
            pltpu.VMEM((128,), jnp.int32),
            pltpu.VMEM((8, 128), jnp.float32),
            pltpu.VMEM((8, 128), jnp.float32),
            pltpu.VMEM((4, 128), jnp.float32),
            pltpu.VMEM_SHARED((128, 128), jnp.float32),
            pltpu.SemaphoreType.DMA,
            pltpu.SemaphoreType.DMA,
        ],
    )


def _tc_mean(x_ref, out_ref, cnt_ref):
    ci = pl.program_id(1)

    @pl.when(ci == 0)
    def _init():
        out_ref[...] = jnp.zeros_like(out_ref)
        cnt_ref[0, 0] = 0.0

    blk = x_ref[0]
    out_ref[0] += jnp.sum(blk, axis=0, keepdims=True)
    valid = jnp.any(blk != 0, axis=-1)
    cnt_ref[0, 0] += jnp.sum(valid.astype(jnp.float32))

    @pl.when(ci == _NC - 1)
    def _finish():
        out_ref[...] = out_ref[...] / cnt_ref[0, 0]


@jax.jit
def kernel(x):
    y_sc = _build_sc()(x).reshape(_KS, _D)
    y_tc = pl.pallas_call(
        _tc_mean,
        grid=(_KT, _NC),
        in_specs=[pl.BlockSpec((1, _CHUNK, _D), lambda b, c: (b + _KS, c, 0))],
        out_specs=pl.BlockSpec((1, 1, _D), lambda b, c: (b, 0, 0)),
        out_shape=jax.ShapeDtypeStruct((_KT, 1, _D), jnp.float32),
        scratch_shapes=[pltpu.SMEM((1, 1), jnp.float32)],
    )(x)
    return jnp.concatenate([y_sc, y_tc[:, 0, :]], axis=0)


# speedup vs baseline: 3.3260x; 1.0021x over previous
import functools

import jax
import jax.numpy as jnp
from jax import lax
from jax.experimental import pallas as pl
from jax.experimental.pallas import tpu as pltpu
from jax.experimental.pallas import tpu_sc as plsc

_B, _N, _D = 16, 4096, 512
_L = 16
_NJ = _D // _L
_R = 64
_TR = _R // 8

_KS = 4
_KT = _B - _KS
_WPS = 32 // _KS
_RPW = _N // _WPS
_NCH = _RPW // _R

_CHUNK = 512
_NSTRM = 2
_NC = _N // (_CHUNK * _NSTRM)


def _sc_mean(x_hbm, out_hbm, buf0, buf1, flag8, part_v, tmp_v, res_v, shared,
             sem0, sem1):
    c = lax.axis_index("c")
    s = lax.axis_index("s")
    g = c * 16 + s
    m = g // _WPS
    q = g % _WPS
    row_base = q * _RPW

    def chunk_copy(ci, buf, sem):
        return pltpu.make_async_copy(
            x_hbm.at[m, pl.ds(row_base + ci * _R, _R)], buf, sem)

    iota = lax.iota(jnp.int32, _L)
    idx8 = (iota % 8) * _L
    zero = jnp.zeros((_L,), jnp.float32)
    init = tuple(zero for _ in range(_NJ + 1))

    def process(buf, carry):
        def tr_body(tr, cr):
            accs = list(cr[:_NJ])
            for rr in range(8):
                nz = None
                for j in range(_NJ):
                    v = buf[tr * 8 + rr, pl.ds(j * _L, _L)]
                    accs[j] = accs[j] + v
                    w = plsc.bitcast(v, jnp.int32)
                    nz = w if nz is None else (nz | w)
                flag8[pl.ds(rr * _L, _L)] = nz << 1
            anyf = None
            for col in range(_L):
                fv = plsc.load_gather(flag8, [idx8 + col])
                anyf = fv if anyf is None else (anyf | fv)
            cnt = cr[_NJ] + jnp.where(anyf != 0, 1.0, 0.0)
            return tuple(accs) + (cnt,)

        return lax.fori_loop(0, _TR, tr_body, carry)

    chunk_copy(0, buf0, sem0).start()

    def pair_body(i, carry):
        chunk_copy(2 * i + 1, buf1, sem1).start()
        chunk_copy(2 * i, buf0, sem0).wait()
        carry = process(buf0, carry)
        chunk_copy(jnp.minimum(2 * i + 2, _NCH - 1), buf0, sem0).start()
        chunk_copy(2 * i + 1, buf1, sem1).wait()
        return process(buf1, carry)

    carry = lax.fori_loop(0, _NCH // 2, pair_body, init)
    chunk_copy(_NCH - 1, buf0, sem0).wait()

    accs, cntv = carry[:_NJ], carry[_NJ]
    for j in range(_NJ):
        part_v[j // 8, pl.ds((j % 8) * _L, _L)] = accs[j]

    flag8[pl.ds(0, _L)] = plsc.bitcast(cntv, jnp.int32)
    tot = None
    for j in range(_L):
        lane = plsc.load_gather(flag8, [jnp.full((_L,), j, jnp.int32)])
        lane = plsc.bitcast(lane, jnp.float32)
        tot = lane if tot is None else tot + lane
    part_v[4, pl.ds(0, _L)] = tot * 0.5

    pltpu.sync_copy(part_v, shared.at[pl.ds(s * 8, 8)])
    plsc.subcore_barrier()

    @pl.when(q == 0)
    def _combine():
        acc = [part_v[j // 8, pl.ds((j % 8) * _L, _L)] for j in range(_NJ)]
        cnt = part_v[4, pl.ds(0, _L)]
        for p in range(1, _WPS):
            pltpu.sync_copy(shared.at[pl.ds((s + p) * 8, 8)], tmp_v)
            for j in range(_NJ):
                acc[j] = acc[j] + tmp_v[j // 8, pl.ds((j % 8) * _L, _L)]
            cnt = cnt + tmp_v[4, pl.ds(0, _L)]
        inv = 1.0 / cnt
        for j in range(_NJ):
            res_v[j // 8, pl.ds((j % 8) * _L, _L)] = acc[j] * inv
        pltpu.sync_copy(res_v, out_hbm.at[m])


@functools.cache
def _build_sc():
    return pl.kernel(
        _sc_mean,
        mesh=plsc.VectorSubcoreMesh(core_axis_name="c", subcore_axis_name="s",
                                    num_cores=2, num_subcores=16),
        out_type=jax.ShapeDtypeStruct((_KS, 4, 128), jnp.float32),
        compiler_params=pltpu.CompilerParams(needs_layout_passes=False),
        scratch_types=[
            pltpu.VMEM((_R, _D), jnp.float32),
            pltpu.VMEM((_R, _D), jnp.float32),
            pltpu.VMEM((128,), jnp.int32),
            pltpu.VMEM((8, 128), jnp.float32),
            pltpu.VMEM((8, 128), jnp.float32),
            pltpu.VMEM((4, 128), jnp.float32),
            pltpu.VMEM_SHARED((128, 128), jnp.float32),
            pltpu.SemaphoreType.DMA,
            pltpu.SemaphoreType.DMA,
        ],
    )


def _tc_mean(xa_ref, xb_ref, out_ref, cnt_ref):
    ci = pl.program_id(1)

    @pl.when(ci == 0)
    def _init():
        out_ref[...] = jnp.zeros_like(out_ref)
        cnt_ref[0, 0] = 0.0

    cnt = cnt_ref[0, 0]
    for blk in (xa_ref[0], xb_ref[0]):
        out_ref[0] += jnp.sum(blk, axis=0, keepdims=True)
        valid = jnp.any(blk != 0, axis=-1)
        cnt += jnp.sum(valid.astype(jnp.float32))
    cnt_ref[0, 0] = cnt

    @pl.when(ci == _NC - 1)
    def _finish():
        out_ref[...] = out_ref[...] / cnt_ref[0, 0]


@jax.jit
def kernel(x):
    y_sc = _build_sc()(x).reshape(_KS, _D)
    y_tc = pl.pallas_call(
        _tc_mean,
        grid=(_KT, _NC),
        in_specs=[
            pl.BlockSpec((1, _CHUNK, _D),
                         lambda b, c: (b + _KS, _NSTRM * c, 0)),
            pl.BlockSpec((1, _CHUNK, _D),
                         lambda b, c: (b + _KS, _NSTRM * c + 1, 0)),
        ],
        out_specs=pl.BlockSpec((1, 1, _D), lambda b, c: (b, 0, 0)),
        out_shape=jax.ShapeDtypeStruct((_KT, 1, _D), jnp.float32),
        scratch_shapes=[pltpu.SMEM((1, 1), jnp.float32)],
    )(x, x)
    return jnp.concatenate([y_sc, y_tc[:, 0, :]], axis=0)
